# baseline (speedup 1.0000x reference)
import functools

import jax
import jax.numpy as jnp
from jax import lax
from jax.experimental import pallas as pl
from jax.experimental.pallas import tpu as pltpu
from jax.sharding import PartitionSpec as P


def _main_kernel(q_ref, keys_ref, uq_ref, sm_ref, s_raw_ref, mloc_ref,
                 zloc_ref, uloc_ref, lsum_ref):
    i = pl.program_id(0)
    nsub = q_ref.shape[0]
    msl, d = keys_ref.shape
    tb = q_ref.shape[2]

    @pl.when(i == 0)
    def _init():
        mloc_ref[...] = jnp.full_like(mloc_ref[...], -jnp.inf)
        zloc_ref[...] = jnp.zeros_like(zloc_ref[...])
        uloc_ref[...] = jnp.zeros_like(uloc_ref[...])
        lsum_ref[...] = jnp.zeros_like(lsum_ref[...])

    keys = keys_ref[...]
    k2 = jnp.sum(keys * keys, axis=1, keepdims=True)
    ksum = jnp.sum(keys, axis=1, keepdims=True)

    for sub in range(nsub):
        q = q_ref[sub]

        qnorm = jnp.sqrt(jnp.sum(q * q, axis=0, keepdims=True))
        qn = q / jnp.maximum(qnorm, 1e-12)

        s = jax.lax.dot_general(keys, qn, (((1,), (0,)), ((), ())),
                                preferred_element_type=jnp.float32)
        s_raw_ref[:, sub * tb:(sub + 1) * tb] = s

        smax = jnp.max(s, axis=0, keepdims=True)
        se = jnp.exp(s - smax)
        sm = se / jnp.sum(se, axis=0, keepdims=True)
        sm_ref[:, sub * tb:(sub + 1) * tb] = sm

        cm = jax.lax.dot_general(keys, sm, (((0,), (0,)), ((), ())),
                                 preferred_element_type=jnp.float32)
        uq_ref[sub, :d, :] = qn
        uq_ref[sub, d:, :] = cm

        ii = jax.lax.broadcasted_iota(jnp.int32, s.shape, 0)
        idx1 = jnp.min(jnp.where(s == smax, ii, msl), axis=0, keepdims=True)
        oh1 = (ii == idx1).astype(jnp.float32)
        s2 = jnp.where(ii == idx1, -jnp.inf, s)
        m2 = jnp.max(s2, axis=0, keepdims=True)
        idx2 = jnp.min(jnp.where(s2 == m2, ii, msl), axis=0, keepdims=True)
        oh2 = (ii == idx2).astype(jnp.float32)

        s_t1 = jnp.sum(oh1 * s, axis=0, keepdims=True)
        s_t2 = jnp.sum(oh2 * s, axis=0, keepdims=True)
        k2_t1 = jnp.sum(oh1 * k2, axis=0, keepdims=True)
        k2_t2 = jnp.sum(oh2 * k2, axis=0, keepdims=True)
        ks_t1 = jnp.sum(oh1 * ksum, axis=0, keepdims=True)
        ks_t2 = jnp.sum(oh2 * ksum, axis=0, keepdims=True)
        qsum = jnp.sum(qn, axis=0, keepdims=True)

        eps = jnp.float32(1e-6)
        dsq_p = 1.0 + k2_t1 - 2.0 * s_t1
        dsq_n = 1.0 + k2_t2 - 2.0 * s_t2
        dp = jnp.sqrt(dsq_p + 2.0 * eps * (qsum - ks_t1) + d * eps * eps)
        dn = jnp.sqrt(dsq_n + 2.0 * eps * (qsum - ks_t2) + d * eps * eps)
        sep_b = jnp.sum(jnp.maximum(dp - dn + 1.0, 0.0))
        comp_b = jnp.sum(dsq_p)
        lsum_ref[...] = lsum_ref[...] + jnp.concatenate(
            [sep_b.reshape(1, 1), comp_b.reshape(1, 1)], axis=1)

        mb = jnp.max(s, axis=1, keepdims=True)
        mnew = jnp.maximum(mloc_ref[...], mb)
        scale = jnp.exp(mloc_ref[...] - mnew)
        e = jnp.exp(s - mnew)
        zloc_ref[...] = zloc_ref[...] * scale + jnp.sum(
            e, axis=1, keepdims=True)
        sel = e * oh1
        du = jax.lax.dot_general(sel, qn, (((1,), (1,)), ((), ())),
                                 preferred_element_type=jnp.float32)
        uloc_ref[...] = uloc_ref[...] * scale + du
        mloc_ref[...] = mnew


def _fin_kernel(n, s_ref, m_ref, z_ref, u_ref, keys_ref, l_ref,
                sq_ref, um_ref, sep_ref, comp_ref):
    d = keys_ref.shape[1]
    sq_ref[...] = jnp.exp(s_ref[...] - m_ref[...]) / z_ref[...]
    upd = u_ref[...] + keys_ref[...]
    un = jnp.sqrt(jnp.sum(upd * upd, axis=1, keepdims=True))
    um_ref[...] = upd / jnp.maximum(un, 1e-12)
    sep_ref[...] = l_ref[:, 0:1] / n
    comp_ref[...] = l_ref[:, 1:2] / (n * d)


def _build_main(bs, c, hw, msl, merge, interpret=False):
    ns = bs * hw
    f32 = jnp.float32
    return pl.pallas_call(
        _main_kernel,
        grid=(bs // merge,),
        in_specs=[
            pl.BlockSpec((merge, c, hw), lambda i: (i, 0, 0)),
            pl.BlockSpec((msl, c), lambda i: (0, 0)),
        ],
        out_specs=[
            pl.BlockSpec((merge, 2 * c, hw), lambda i: (i, 0, 0)),
            pl.BlockSpec((msl, merge * hw), lambda i: (0, i)),
            pl.BlockSpec((msl, merge * hw), lambda i: (0, i)),
            pl.BlockSpec((msl, 1), lambda i: (0, 0)),
            pl.BlockSpec((msl, 1), lambda i: (0, 0)),
            pl.BlockSpec((msl, c), lambda i: (0, 0)),
            pl.BlockSpec((1, 2), lambda i: (0, 0)),
        ],
        out_shape=[
            jax.ShapeDtypeStruct((bs, 2 * c, hw), f32),
            jax.ShapeDtypeStruct((msl, ns), f32),
            jax.ShapeDtypeStruct((msl, ns), f32),
            jax.ShapeDtypeStruct((msl, 1), f32),
            jax.ShapeDtypeStruct((msl, 1), f32),
            jax.ShapeDtypeStruct((msl, c), f32),
            jax.ShapeDtypeStruct((1, 2), f32),
        ],
        compiler_params=pltpu.CompilerParams(
            dimension_semantics=("arbitrary",)),
        interpret=interpret,
    )


def _build_fin(n, ns, c, msl, interpret=False):
    f32 = jnp.float32
    return pl.pallas_call(
        functools.partial(_fin_kernel, n),
        out_shape=[
            jax.ShapeDtypeStruct((msl, ns), f32),
            jax.ShapeDtypeStruct((msl, c), f32),
            jax.ShapeDtypeStruct((1, 1), f32),
            jax.ShapeDtypeStruct((1, 1), f32),
        ],
        interpret=interpret,
    )


def _shard_fn(n, merge, interpret, q3, keys):
    bs, c, hw = q3.shape
    msl = keys.shape[0]
    uq, sm_t, s_raw, mloc, zloc, uloc, lsum = _build_main(
        bs, c, hw, msl, merge, interpret)(q3, keys)
    mg = lax.pmax(mloc, "x")
    coef = jnp.exp(mloc - mg)
    zg = lax.psum(zloc * coef, "x")
    ug = lax.psum(uloc * coef, "x")
    lg = lax.psum(lsum, "x")
    sq_t, um, sep, comp = _build_fin(n, bs * hw, c, msl, interpret)(
        s_raw, mg, zg, ug, keys, lg)
    return uq, um, sq_t.T, sm_t.T, sep, comp


def kernel(query, keys):
    b, c, h, w = query.shape
    hw = h * w
    devs = jax.devices()
    nd = len(devs)
    while b % nd:
        nd -= 1
    mesh = jax.sharding.Mesh(devs[:nd], ("x",))
    bs = b // nd
    merge = 4
    while bs % merge:
        merge -= 1
    fn = jax.shard_map(
        functools.partial(_shard_fn, b * hw, merge, False),
        mesh=mesh,
        in_specs=(P("x", None, None), P(None, None)),
        out_specs=(P("x", None, None), P(None, None), P("x", None),
                   P("x", None), P(None, None), P(None, None)),
        check_vma=False,
    )
    q3 = query.reshape(b, c, hw)
    uq, um, sq, sm, sep, comp = fn(q3, keys)
    updated_query = uq.reshape(b, 2 * c, h, w)
    return (updated_query, um, sq, sm, sep[0, 0], comp[0, 0])
